# Initial kernel scaffold; baseline (speedup 1.0000x reference)
import jax
import jax.numpy as jnp
from jax.experimental import pallas as pl


def kernel(x, edge_index, edge_attr, y, W_fuse, b_fuse, gamma, beta, W_ih0, W_hh0, b_ih0, b_hh0, W_ih1, W_hh1, b_ih1, b_hh1, W_fc, b_fc):
    raise NotImplementedError("write your pallas kernel here")


# speedup vs baseline: 2.8347x; 2.8347x over previous
import math

import jax
import jax.numpy as jnp
from jax.experimental import pallas as pl
from jax.experimental.pallas import tpu as pltpu

N = 2048
SLOTS = 4
F = 256
C = 128
E = 8
MAXLEN = 512
SLAB = MAXLEN + 8
TPAD = 528
GPAD = N + MAXLEN
OPAD = N + TPAD


def _gru_gates(gi, gh, h):
    r = jax.nn.sigmoid(gi[:, :C] + gh[:, :C])
    z = jax.nn.sigmoid(gi[:, C:2 * C] + gh[:, C:2 * C])
    n = jnp.tanh(gi[:, 2 * C:] + r * gh[:, 2 * C:])
    return (1.0 - z) * n + z * h


def _graph_rnn_kernel(starts_ref, sal_ref, counts_ref,
                      audio_ref, video_ref, avf_ref, vvf_ref,
                      rvec_ref,
                      Wfa_ref, Wfv_ref, bf_ref, gamma_ref, beta_ref,
                      Wi0_ref, Wh0_ref, bi0_ref, bh0_ref,
                      Wi1_ref, Wh1_ref, bi1_ref, bh1_ref,
                      Wfc_ref, bfc_ref,
                      out_ref,
                      gi0_ref, packed_ref, hist_ref, tok_ref):
    a = avf_ref[...]
    v = vvf_ref[...]
    dot = jnp.sum(a * v, axis=1, keepdims=True)
    na = jnp.maximum(jnp.sqrt(jnp.sum(a * a, axis=1, keepdims=True)), 1e-8)
    nb = jnp.maximum(jnp.sqrt(jnp.sum(v * v, axis=1, keepdims=True)), 1e-8)
    sim = dot / (na * nb)
    audio = audio_ref[...] * sim
    g = (jnp.dot(audio, Wfa_ref[...], preferred_element_type=jnp.float32)
         + jnp.dot(video_ref[...], Wfv_ref[...], preferred_element_type=jnp.float32)
         + bf_ref[...])
    g = g * (gamma_ref[...] * (1.0 / math.sqrt(1.0 + 1e-5))) + beta_ref[...]
    g = jnp.maximum(g, 0.0)
    gi0_ref[:N, :] = (jnp.dot(g, Wi0_ref[...], preferred_element_type=jnp.float32)
                      + bi0_ref[...])
    gi0_ref[N:, :] = jnp.zeros((GPAD - N, 3 * C), jnp.float32)

    for e in range(E):
        base = sal_ref[e] * 8
        packed_ref[:SLAB, e, :] = gi0_ref[pl.ds(base, SLAB), :]

    tmax = counts_ref[0] + (starts_ref[0] - sal_ref[0] * 8)
    for e in range(1, E):
        tmax = jnp.maximum(tmax, counts_ref[e] + (starts_ref[e] - sal_ref[e] * 8))

    rv = rvec_ref[...]

    def body(t, carry):
        h0, h1 = carry
        keep = t >= rv
        xg = packed_ref[pl.ds(t, 1), :, :].reshape(E, 3 * C)
        gh0 = (jnp.dot(h0, Wh0_ref[...], preferred_element_type=jnp.float32)
               + bh0_ref[...])
        h0n = jnp.where(keep, _gru_gates(xg, gh0, h0), 0.0)
        gi1 = (jnp.dot(h0n, Wi1_ref[...], preferred_element_type=jnp.float32)
               + bi1_ref[...])
        gh1 = (jnp.dot(h1, Wh1_ref[...], preferred_element_type=jnp.float32)
               + bh1_ref[...])
        h1n = jnp.where(keep, _gru_gates(gi1, gh1, h1), 0.0)
        hist_ref[pl.ds(t, 1), :, :] = h1n.reshape(1, E, C)
        return (h0n, h1n)

    h_init = jnp.zeros((E, C), jnp.float32)
    jax.lax.fori_loop(0, tmax, body, (h_init, h_init))

    rows = jax.lax.broadcasted_iota(jnp.int32, (SLAB, 1), 0)
    for e in range(E):
        base = sal_ref[e] * 8
        r_e = starts_ref[e] - base
        m = (rows >= r_e) & (rows < r_e + counts_ref[e])
        cur = tok_ref[pl.ds(base, SLAB), :]
        tok_ref[pl.ds(base, SLAB), :] = jnp.where(m, hist_ref[:SLAB, e, :], cur)

    out_ref[...] = (jnp.dot(tok_ref[:N, :], Wfc_ref[...],
                            preferred_element_type=jnp.float32)
                    + bfc_ref[...])


def _build(interpret=False):
    return pl.pallas_call(
        _graph_rnn_kernel,
        out_shape=jax.ShapeDtypeStruct((N, 2), jnp.float32),
        in_specs=(
            [pl.BlockSpec(memory_space=pltpu.SMEM)] * 3
            + [pl.BlockSpec(memory_space=pltpu.VMEM)] * 20
        ),
        out_specs=pl.BlockSpec(memory_space=pltpu.VMEM),
        scratch_shapes=[
            pltpu.VMEM((GPAD, 3 * C), jnp.float32),
            pltpu.VMEM((TPAD, E, 3 * C), jnp.float32),
            pltpu.VMEM((TPAD, E, C), jnp.float32),
            pltpu.VMEM((OPAD, C), jnp.float32),
        ],
        interpret=interpret,
    )


def kernel(x, edge_index, edge_attr, y, W_fuse, b_fuse, gamma, beta,
           W_ih0, W_hh0, b_ih0, b_hh0, W_ih1, W_hh1, b_ih1, b_hh1,
           W_fc, b_fc):
    ent = y[:, -1].astype(jnp.int32)
    counts = jnp.bincount(ent, length=E).astype(jnp.int32)
    starts = jnp.concatenate(
        [jnp.zeros((1,), jnp.int32), jnp.cumsum(counts)[:-1].astype(jnp.int32)])
    sal = starts // 8
    rvec = (starts - sal * 8).reshape(E, 1)
    call = _build()
    return call(
        starts, sal, counts,
        x[:, 0, :], x[:, 1, :], x[:, 2, :], x[:, 3, :],
        rvec,
        W_fuse[:, :F].T, W_fuse[:, F:].T, b_fuse[None, :],
        gamma[None, :], beta[None, :],
        W_ih0.T, W_hh0.T, b_ih0[None, :], b_hh0[None, :],
        W_ih1.T, W_hh1.T, b_ih1[None, :], b_hh1[None, :],
        W_fc.T, b_fc[None, :])


# speedup vs baseline: 3.6540x; 1.2890x over previous
import math

import jax
import jax.numpy as jnp
from jax.experimental import pallas as pl
from jax.experimental.pallas import tpu as pltpu

N = 2048
SLOTS = 4
F = 256
C = 128
E = 8
MAXLEN = 512
SLAB = MAXLEN + 8
TPAD = 528
GPAD = N + MAXLEN
OPAD = N + TPAD


def _gru_gates(gi, gh, h):
    r = jax.nn.sigmoid(gi[:, :C] + gh[:, :C])
    z = jax.nn.sigmoid(gi[:, C:2 * C] + gh[:, C:2 * C])
    n = jnp.tanh(gi[:, 2 * C:] + r * gh[:, 2 * C:])
    return (1.0 - z) * n + z * h


def _graph_rnn_kernel(starts_ref, sal_ref, counts_ref,
                      audio_ref, video_ref, avf_ref, vvf_ref,
                      rvec_ref,
                      Wfa_ref, Wfv_ref, bf_ref, gamma_ref, beta_ref,
                      Wi0_ref, Wh0_ref, bi0_ref, bh0_ref,
                      Wi1_ref, Wh1_ref, bi1_ref, bh1_ref,
                      Wfc_ref, bfc_ref,
                      out_ref,
                      gi0_ref, packed_ref, hist_ref, tok_ref):
    a = avf_ref[...]
    v = vvf_ref[...]
    dot = jnp.sum(a * v, axis=1, keepdims=True)
    na = jnp.maximum(jnp.sqrt(jnp.sum(a * a, axis=1, keepdims=True)), 1e-8)
    nb = jnp.maximum(jnp.sqrt(jnp.sum(v * v, axis=1, keepdims=True)), 1e-8)
    sim = dot / (na * nb)
    audio = audio_ref[...] * sim
    g = (jnp.dot(audio, Wfa_ref[...], preferred_element_type=jnp.float32)
         + jnp.dot(video_ref[...], Wfv_ref[...], preferred_element_type=jnp.float32)
         + bf_ref[...])
    g = g * (gamma_ref[...] * (1.0 / math.sqrt(1.0 + 1e-5))) + beta_ref[...]
    g = jnp.maximum(g, 0.0)
    gi0_ref[:N, :] = (jnp.dot(g, Wi0_ref[...], preferred_element_type=jnp.float32)
                      + bi0_ref[...])
    gi0_ref[N:, :] = jnp.zeros((GPAD - N, 3 * C), jnp.float32)

    for e in range(E):
        base = sal_ref[e] * 8
        packed_ref[:SLAB, e, :] = gi0_ref[pl.ds(base, SLAB), :]

    tmax = counts_ref[0] + (starts_ref[0] - sal_ref[0] * 8)
    for e in range(1, E):
        tmax = jnp.maximum(tmax, counts_ref[e] + (starts_ref[e] - sal_ref[e] * 8))

    rv = rvec_ref[...]

    def body(t, carry):
        h0, h1, y0p = carry
        keep0 = t >= rv
        xg = packed_ref[pl.ds(t, 1), :, :].reshape(E, 3 * C)
        gh0 = (jnp.dot(h0, Wh0_ref[...], preferred_element_type=jnp.float32)
               + bh0_ref[...])
        h0n = jnp.where(keep0, _gru_gates(xg, gh0, h0), 0.0)

        keep1 = (t - 1) >= rv
        gi1 = (jnp.dot(y0p, Wi1_ref[...], preferred_element_type=jnp.float32)
               + bi1_ref[...])
        gh1 = (jnp.dot(h1, Wh1_ref[...], preferred_element_type=jnp.float32)
               + bh1_ref[...])
        h1n = jnp.where(keep1, _gru_gates(gi1, gh1, h1), 0.0)
        hist_ref[pl.ds(jnp.maximum(t - 1, 0), 1), :, :] = h1n.reshape(1, E, C)
        return (h0n, h1n, h0n)

    h_init = jnp.zeros((E, C), jnp.float32)
    jax.lax.fori_loop(0, tmax + 1, body, (h_init, h_init, h_init))

    rows = jax.lax.broadcasted_iota(jnp.int32, (SLAB, 1), 0)
    for e in range(E):
        base = sal_ref[e] * 8
        r_e = starts_ref[e] - base
        m = (rows >= r_e) & (rows < r_e + counts_ref[e])
        cur = tok_ref[pl.ds(base, SLAB), :]
        tok_ref[pl.ds(base, SLAB), :] = jnp.where(m, hist_ref[:SLAB, e, :], cur)

    out_ref[...] = (jnp.dot(tok_ref[:N, :], Wfc_ref[...],
                            preferred_element_type=jnp.float32)
                    + bfc_ref[...])


def _build(interpret=False):
    return pl.pallas_call(
        _graph_rnn_kernel,
        out_shape=jax.ShapeDtypeStruct((N, 2), jnp.float32),
        in_specs=(
            [pl.BlockSpec(memory_space=pltpu.SMEM)] * 3
            + [pl.BlockSpec(memory_space=pltpu.VMEM)] * 20
        ),
        out_specs=pl.BlockSpec(memory_space=pltpu.VMEM),
        scratch_shapes=[
            pltpu.VMEM((GPAD, 3 * C), jnp.float32),
            pltpu.VMEM((TPAD, E, 3 * C), jnp.float32),
            pltpu.VMEM((TPAD, E, C), jnp.float32),
            pltpu.VMEM((OPAD, C), jnp.float32),
        ],
        interpret=interpret,
    )


def kernel(x, edge_index, edge_attr, y, W_fuse, b_fuse, gamma, beta,
           W_ih0, W_hh0, b_ih0, b_hh0, W_ih1, W_hh1, b_ih1, b_hh1,
           W_fc, b_fc):
    ent = y[:, -1].astype(jnp.int32)
    counts = jnp.bincount(ent, length=E).astype(jnp.int32)
    starts = jnp.concatenate(
        [jnp.zeros((1,), jnp.int32), jnp.cumsum(counts)[:-1].astype(jnp.int32)])
    sal = starts // 8
    rvec = (starts - sal * 8).reshape(E, 1)
    call = _build()
    return call(
        starts, sal, counts,
        x[:, 0, :], x[:, 1, :], x[:, 2, :], x[:, 3, :],
        rvec,
        W_fuse[:, :F].T, W_fuse[:, F:].T, b_fuse[None, :],
        gamma[None, :], beta[None, :],
        W_ih0.T, W_hh0.T, b_ih0[None, :], b_hh0[None, :],
        W_ih1.T, W_hh1.T, b_ih1[None, :], b_hh1[None, :],
        W_fc.T, b_fc[None, :])


# speedup vs baseline: 4.5908x; 1.2564x over previous
import math

import jax
import jax.numpy as jnp
from jax.experimental import pallas as pl
from jax.experimental.pallas import tpu as pltpu

N = 2048
SLOTS = 4
F = 256
C = 128
E = 8
MAXLEN = 512
SLAB = MAXLEN + 8
TPAD = 528
GPAD = N + MAXLEN
OPAD = N + TPAD


def _dot_t(a, w):
    return jax.lax.dot_general(a, w, (((1,), (1,)), ((), ())),
                               preferred_element_type=jnp.float32)


def _dot(a, w):
    return jnp.dot(a, w, preferred_element_type=jnp.float32)


def _gru_gates(gi, gh, h):
    r = jax.nn.sigmoid(gi[:, :C] + gh[:, :C])
    z = jax.nn.sigmoid(gi[:, C:2 * C] + gh[:, C:2 * C])
    n = jnp.tanh(gi[:, 2 * C:] + r * gh[:, 2 * C:])
    return (1.0 - z) * n + z * h


def _graph_rnn_kernel(starts_ref, sal_ref, counts_ref,
                      xr_ref, rvec_ref,
                      Wfuse_ref, bf_ref, gamma_ref, beta_ref,
                      Wi0_ref, Wh0_ref, bi0_ref, bh0_ref,
                      Wi1_ref, Wh1_ref, bi1_ref, bh1_ref,
                      Wfc_ref, bfc_ref,
                      out_ref,
                      gi0_ref, packed_ref, hist_ref, tok_ref,
                      whT0_ref, wiT1_ref, whT1_ref):
    whT0_ref[...] = Wh0_ref[...].T
    wiT1_ref[...] = Wi1_ref[...].T
    whT1_ref[...] = Wh1_ref[...].T

    a = xr_ref[:, 2 * F:3 * F]
    v = xr_ref[:, 3 * F:]
    dot = jnp.sum(a * v, axis=1, keepdims=True)
    na = jnp.maximum(jnp.sqrt(jnp.sum(a * a, axis=1, keepdims=True)), 1e-8)
    nb = jnp.maximum(jnp.sqrt(jnp.sum(v * v, axis=1, keepdims=True)), 1e-8)
    sim = dot / (na * nb)
    audio = xr_ref[:, :F] * sim
    g = (_dot_t(audio, Wfuse_ref[:, :F])
         + _dot_t(xr_ref[:, F:2 * F], Wfuse_ref[:, F:])
         + bf_ref[...])
    g = g * (gamma_ref[...] * (1.0 / math.sqrt(1.0 + 1e-5))) + beta_ref[...]
    g = jnp.maximum(g, 0.0)
    gi0_ref[:N, :] = _dot_t(g, Wi0_ref[...]) + bi0_ref[...]
    gi0_ref[N:, :] = jnp.zeros((GPAD - N, 3 * C), jnp.float32)

    for e in range(E):
        base = sal_ref[e] * 8
        packed_ref[:SLAB, e, :] = gi0_ref[pl.ds(base, SLAB), :]

    tmax = counts_ref[0] + (starts_ref[0] - sal_ref[0] * 8)
    for e in range(1, E):
        tmax = jnp.maximum(tmax, counts_ref[e] + (starts_ref[e] - sal_ref[e] * 8))

    rv = rvec_ref[...]

    def body(t, carry):
        h0, h1, y0p = carry
        keep0 = t >= rv
        xg = packed_ref[pl.ds(t, 1), :, :].reshape(E, 3 * C)
        gh0 = _dot(h0, whT0_ref[...]) + bh0_ref[...]
        h0n = jnp.where(keep0, _gru_gates(xg, gh0, h0), 0.0)

        keep1 = (t - 1) >= rv
        gi1 = _dot(y0p, wiT1_ref[...]) + bi1_ref[...]
        gh1 = _dot(h1, whT1_ref[...]) + bh1_ref[...]
        h1n = jnp.where(keep1, _gru_gates(gi1, gh1, h1), 0.0)
        hist_ref[pl.ds(jnp.maximum(t - 1, 0), 1), :, :] = h1n.reshape(1, E, C)
        return (h0n, h1n, h0n)

    h_init = jnp.zeros((E, C), jnp.float32)
    jax.lax.fori_loop(0, tmax + 1, body, (h_init, h_init, h_init))

    rows = jax.lax.broadcasted_iota(jnp.int32, (SLAB, 1), 0)
    for e in range(E):
        base = sal_ref[e] * 8
        r_e = starts_ref[e] - base
        m = (rows >= r_e) & (rows < r_e + counts_ref[e])
        cur = tok_ref[pl.ds(base, SLAB), :]
        tok_ref[pl.ds(base, SLAB), :] = jnp.where(m, hist_ref[:SLAB, e, :], cur)

    out_ref[...] = _dot_t(tok_ref[:N, :], Wfc_ref[...]) + bfc_ref[...]


def _build(interpret=False):
    return pl.pallas_call(
        _graph_rnn_kernel,
        out_shape=jax.ShapeDtypeStruct((N, 2), jnp.float32),
        in_specs=(
            [pl.BlockSpec(memory_space=pltpu.SMEM)] * 3
            + [pl.BlockSpec(memory_space=pltpu.VMEM)] * 16
        ),
        out_specs=pl.BlockSpec(memory_space=pltpu.VMEM),
        scratch_shapes=[
            pltpu.VMEM((GPAD, 3 * C), jnp.float32),
            pltpu.VMEM((TPAD, E, 3 * C), jnp.float32),
            pltpu.VMEM((TPAD, E, C), jnp.float32),
            pltpu.VMEM((OPAD, C), jnp.float32),
            pltpu.VMEM((C, 3 * C), jnp.float32),
            pltpu.VMEM((C, 3 * C), jnp.float32),
            pltpu.VMEM((C, 3 * C), jnp.float32),
        ],
        interpret=interpret,
    )


def kernel(x, edge_index, edge_attr, y, W_fuse, b_fuse, gamma, beta,
           W_ih0, W_hh0, b_ih0, b_hh0, W_ih1, W_hh1, b_ih1, b_hh1,
           W_fc, b_fc):
    ent = y[:, -1].astype(jnp.int32)
    counts = jnp.sum(
        (ent[:, None] == jnp.arange(E, dtype=jnp.int32)[None, :]).astype(jnp.int32),
        axis=0)
    starts = jnp.concatenate(
        [jnp.zeros((1,), jnp.int32), jnp.cumsum(counts)[:-1].astype(jnp.int32)])
    sal = starts // 8
    rvec = (starts - sal * 8).reshape(E, 1)
    call = _build()
    return call(
        starts, sal, counts,
        x.reshape(N, SLOTS * F), rvec,
        W_fuse, b_fuse[None, :], gamma[None, :], beta[None, :],
        W_ih0, W_hh0, b_ih0[None, :], b_hh0[None, :],
        W_ih1, W_hh1, b_ih1[None, :], b_hh1[None, :],
        W_fc, b_fc[None, :])
